# baseline (speedup 1.0000x reference)
import functools

import jax
import jax.numpy as jnp
from jax import lax
from jax.experimental import pallas as pl
from jax.experimental.pallas import tpu as pltpu
from jax.experimental.pallas import tpu_sc as plsc

B, L, D = 1024, 200, 128
N = B * L
NC, NS = 2, 16
NW = NC * NS
RPW = N // NW
C = 64
NCHUNK = RPW // C
SEG = D // 16
RING = 4

_mesh = plsc.VectorSubcoreMesh(core_axis_name="c", subcore_axis_name="s")


@functools.partial(
    pl.kernel,
    mesh=_mesh,
    out_type=jax.ShapeDtypeStruct((N, D), jnp.float32),
    scratch_types=(
        [pltpu.VMEM((6, C), jnp.int32) for _ in range(RING)]
        + [pltpu.VMEM((C, D), jnp.float32) for _ in range(RING)]
        + [pltpu.SemaphoreType.DMA for _ in range(3 * RING)]
    ),
)
def _embed_sum(tok_t, cnt_t, val_t, io_t, pos_t, gas_t, idx_hbm, out_hbm,
               ix0, ix1, ix2, ix3, ac0, ac1, ac2, ac3,
               sg0, sg1, sg2, sg3, so0, so1, so2, so3, si0, si1, si2, si3):
    ixb = [ix0, ix1, ix2, ix3]
    acc = [ac0, ac1, ac2, ac3]
    sG = [sg0, sg1, sg2, sg3]
    sO = [so0, so1, so2, so3]
    sI = [si0, si1, si2, si3]
    tables = [tok_t, cnt_t, val_t, io_t, pos_t, gas_t]

    wid = lax.axis_index("s") * NC + lax.axis_index("c")
    kbase = wid * NCHUNK
    rbase = wid * RPW
    zv = jnp.zeros((16,), jnp.float32)

    for j in range(RING - 1):
        pltpu.async_copy(idx_hbm.at[kbase + j], ixb[j], sI[j])

    def visit(gg, j):
        @pl.when(gg >= RING)
        def _():
            pltpu.make_async_copy(acc[j], out_hbm.at[pl.ds(0, C)], sO[j]).wait()

        def zrow(r, z):
            for s in range(SEG):
                acc[j][r, pl.ds(s * 16, 16)] = zv
            return z
        lax.fori_loop(0, C, zrow, 0)

        pltpu.make_async_copy(idx_hbm.at[kbase], ixb[j], sI[j]).wait()

        for t in range(6):
            pltpu.async_copy(tables[t].at[ixb[j].at[t]], acc[j], sG[j],
                             add=True)

        jp = (j - 1) % RING

        @pl.when(gg >= 1)
        def _():
            for _ in range(6):
                pltpu.make_async_copy(out_hbm.at[pl.ds(0, C)], acc[jp],
                                      sG[jp]).wait()
            pltpu.async_copy(acc[jp],
                             out_hbm.at[pl.ds(rbase + (gg - 1) * C, C)],
                             sO[jp])

        @pl.when(gg + RING - 1 < NCHUNK)
        def _():
            pltpu.async_copy(idx_hbm.at[kbase + gg + RING - 1],
                             ixb[(j + RING - 1) % RING],
                             sI[(j + RING - 1) % RING])

    def outer(g4, carry):
        for j in range(RING):
            visit(g4 * RING + j, j)
        return carry

    lax.fori_loop(0, NCHUNK // RING, outer, 0)

    jl = (NCHUNK - 1) % RING
    for _ in range(6):
        pltpu.make_async_copy(out_hbm.at[pl.ds(0, C)], acc[jl], sG[jl]).wait()
    pltpu.async_copy(acc[jl],
                     out_hbm.at[pl.ds(rbase + (NCHUNK - 1) * C, C)], sO[jl])
    for j in range(RING):
        pltpu.make_async_copy(acc[j], out_hbm.at[pl.ds(0, C)], sO[j]).wait()


def kernel(input_ids, counts, values, io_flags, positions, gas_fee,
           token_table, count_table, value_table, position_table,
           io_table, gas_table):
    flat = lambda a: a.reshape(N).astype(jnp.int32)
    idx = jnp.stack([flat(input_ids), flat(counts), flat(values),
                     flat(io_flags), flat(positions), flat(gas_fee)])
    idx = idx.reshape(6, NW * NCHUNK, C).transpose(1, 0, 2)
    out = _embed_sum(token_table, count_table, value_table, io_table,
                     position_table, gas_table, idx)
    return out.reshape(B, L, D)


# speedup vs baseline: 12.7214x; 12.7214x over previous
import functools

import jax
import jax.numpy as jnp
from jax import lax
from jax.experimental import pallas as pl
from jax.experimental.pallas import tpu as pltpu
from jax.experimental.pallas import tpu_sc as plsc

B, L, D = 1024, 200, 128
N = B * L
NC, NS = 2, 16
NW = NC * NS
RPW = N // NW
C = 64
NCHUNK = RPW // C
SEG = D // 16
RING = 4
CSL = C // 16

_mesh = plsc.VectorSubcoreMesh(core_axis_name="c", subcore_axis_name="s")


def _build_tables_body(cnt_ref, val_ref, gas_ref, io_ref, pos_ref,
                       t1_ref, t2_ref):
    cnt = cnt_ref[...]
    val = val_ref[...]
    gas = gas_ref[...]
    t1 = cnt[:, None, None, :] + val[None, :, None, :] + gas[None, None, :, :]
    t1_ref[...] = t1.reshape(15 * 15 * 15, D)
    t2 = io_ref[...][:, None, :] + pos_ref[...][None, :, :]
    t2_ref[...] = t2.reshape(3 * 200, D)


def _build_tables(cnt, val, gas, io, pos):
    return pl.pallas_call(
        _build_tables_body,
        out_shape=(jax.ShapeDtypeStruct((3375, D), jnp.float32),
                   jax.ShapeDtypeStruct((600, D), jnp.float32)),
    )(cnt, val, gas, io, pos)


@functools.partial(
    pl.kernel,
    mesh=_mesh,
    out_type=jax.ShapeDtypeStruct((N, D), jnp.float32),
    scratch_types=(
        [pltpu.VMEM((6, C), jnp.int32) for _ in range(RING)]
        + [pltpu.VMEM((C, D), jnp.float32) for _ in range(RING)]
        + [pltpu.SemaphoreType.DMA for _ in range(3 * RING)]
    ),
)
def _embed_sum(tok_t, t1_t, t2_t, idx_hbm, out_hbm,
               ix0, ix1, ix2, ix3, ac0, ac1, ac2, ac3,
               sg0, sg1, sg2, sg3, so0, so1, so2, so3, si0, si1, si2, si3):
    ixb = [ix0, ix1, ix2, ix3]
    acc = [ac0, ac1, ac2, ac3]
    sG = [sg0, sg1, sg2, sg3]
    sO = [so0, so1, so2, so3]
    sI = [si0, si1, si2, si3]
    tables = [tok_t, t1_t, t2_t]

    wid = lax.axis_index("s") * NC + lax.axis_index("c")
    kbase = wid * NCHUNK
    rbase = wid * RPW
    zv = jnp.zeros((16,), jnp.float32)

    for j in range(RING - 1):
        pltpu.async_copy(idx_hbm.at[kbase + j], ixb[j], sI[j])

    def visit(gg, j):
        @pl.when(gg >= RING)
        def _():
            pltpu.make_async_copy(acc[j], out_hbm.at[pl.ds(0, C)], sO[j]).wait()

        def zrow(r, z):
            for s in range(SEG):
                acc[j][r, pl.ds(s * 16, 16)] = zv
            return z
        lax.fori_loop(0, C, zrow, 0)

        pltpu.make_async_copy(idx_hbm.at[kbase], ixb[j], sI[j]).wait()

        for s in range(CSL):
            cs = pl.ds(s * 16, 16)
            cv = ixb[j][1, cs] * 15 + ixb[j][2, cs]
            ixb[j][1, cs] = cv * 15 + ixb[j][5, cs]
            ixb[j][3, cs] = ixb[j][3, cs] * 200 + ixb[j][4, cs]

        for t, row in ((0, 0), (1, 1), (2, 3)):
            pltpu.async_copy(tables[t].at[ixb[j].at[row]], acc[j], sG[j],
                             add=True)

        jp = (j - 1) % RING

        @pl.when(gg >= 1)
        def _():
            for _ in range(3):
                pltpu.make_async_copy(out_hbm.at[pl.ds(0, C)], acc[jp],
                                      sG[jp]).wait()
            pltpu.async_copy(acc[jp],
                             out_hbm.at[pl.ds(rbase + (gg - 1) * C, C)],
                             sO[jp])

        @pl.when(gg + RING - 1 < NCHUNK)
        def _():
            pltpu.async_copy(idx_hbm.at[kbase + gg + RING - 1],
                             ixb[(j + RING - 1) % RING],
                             sI[(j + RING - 1) % RING])

    def outer(g4, carry):
        for j in range(RING):
            visit(g4 * RING + j, j)
        return carry

    lax.fori_loop(0, NCHUNK // RING, outer, 0)

    jl = (NCHUNK - 1) % RING
    for _ in range(3):
        pltpu.make_async_copy(out_hbm.at[pl.ds(0, C)], acc[jl], sG[jl]).wait()
    pltpu.async_copy(acc[jl],
                     out_hbm.at[pl.ds(rbase + (NCHUNK - 1) * C, C)], sO[jl])
    for j in range(RING):
        pltpu.make_async_copy(acc[j], out_hbm.at[pl.ds(0, C)], sO[j]).wait()


def kernel(input_ids, counts, values, io_flags, positions, gas_fee,
           token_table, count_table, value_table, position_table,
           io_table, gas_table):
    t1, t2 = _build_tables(count_table, value_table, gas_table,
                           io_table, position_table)
    flat = lambda a: a.reshape(N).astype(jnp.int32)
    idx = jnp.stack([flat(input_ids), flat(counts), flat(values),
                     flat(io_flags), flat(positions), flat(gas_fee)])
    idx = idx.reshape(6, NW * NCHUNK, C).transpose(1, 0, 2)
    out = _embed_sum(token_table, t1, t2, idx)
    return out.reshape(B, L, D)


# speedup vs baseline: 14.5060x; 1.1403x over previous
import functools

import jax
import jax.numpy as jnp
from jax import lax
from jax.experimental import pallas as pl
from jax.experimental.pallas import tpu as pltpu
from jax.experimental.pallas import tpu_sc as plsc

B, L, D = 1024, 200, 128
N = B * L
NC, NS = 2, 16
NW = NC * NS
RPW = N // NW
C = 64
NCHUNK = RPW // C
SEG = D // 16
RING = 4
CSL = C // 16

_mesh = plsc.VectorSubcoreMesh(core_axis_name="c", subcore_axis_name="s")


def _build_tables_body(cnt_ref, val_ref, gas_ref, io_ref, pos_ref,
                       t1_ref, t2_ref):
    cnt = cnt_ref[...]
    val = val_ref[...]
    gas = gas_ref[...]
    t1 = cnt[:, None, None, :] + val[None, :, None, :] + gas[None, None, :, :]
    t1_ref[...] = t1.reshape(15 * 15 * 15, D)
    t2 = io_ref[...][:, None, :] + pos_ref[...][None, :, :]
    t2_ref[...] = t2.reshape(3 * 200, D)


def _build_tables(cnt, val, gas, io, pos):
    return pl.pallas_call(
        _build_tables_body,
        out_shape=(jax.ShapeDtypeStruct((3375, D), jnp.float32),
                   jax.ShapeDtypeStruct((600, D), jnp.float32)),
    )(cnt, val, gas, io, pos)


@functools.partial(
    pl.kernel,
    mesh=_mesh,
    out_type=jax.ShapeDtypeStruct((N, D), jnp.float32),
    scratch_types=(
        [pltpu.VMEM((6, C), jnp.int32) for _ in range(RING)]
        + [pltpu.VMEM((C, D), jnp.float32) for _ in range(RING)]
        + [pltpu.SemaphoreType.DMA for _ in range(3 * RING)]
    ),
)
def _embed_sum(tok_t, t1_t, t2_t, itok, icnt, ival, iio, ipos, igas, out_hbm,
               ix0, ix1, ix2, ix3, ac0, ac1, ac2, ac3,
               sg0, sg1, sg2, sg3, so0, so1, so2, so3, si0, si1, si2, si3):
    idxs = [itok, icnt, ival, iio, ipos, igas]
    ixb = [ix0, ix1, ix2, ix3]
    acc = [ac0, ac1, ac2, ac3]
    sG = [sg0, sg1, sg2, sg3]
    sO = [so0, so1, so2, so3]
    sI = [si0, si1, si2, si3]
    tables = [tok_t, t1_t, t2_t]

    wid = lax.axis_index("s") * NC + lax.axis_index("c")
    rbase = wid * RPW
    zv = jnp.zeros((16,), jnp.float32)

    def fetch_idx(gg, j):
        for t in range(6):
            pltpu.async_copy(idxs[t].at[pl.ds(rbase + gg * C, C)],
                             ixb[j].at[t], sI[j])

    for j in range(RING - 1):
        fetch_idx(j, j)

    def visit(gg, j):
        @pl.when(gg >= RING)
        def _():
            pltpu.make_async_copy(acc[j], out_hbm.at[pl.ds(0, C)], sO[j]).wait()

        def zrow(r, z):
            for s in range(SEG):
                acc[j][r, pl.ds(s * 16, 16)] = zv
            return z
        lax.fori_loop(0, C, zrow, 0)

        for t in range(6):
            pltpu.make_async_copy(idxs[0].at[pl.ds(0, C)], ixb[j].at[t],
                                  sI[j]).wait()

        for s in range(CSL):
            cs = pl.ds(s * 16, 16)
            cv = ixb[j][1, cs] * 15 + ixb[j][2, cs]
            ixb[j][1, cs] = cv * 15 + ixb[j][5, cs]
            ixb[j][3, cs] = ixb[j][3, cs] * 200 + ixb[j][4, cs]

        for t, row in ((0, 0), (1, 1), (2, 3)):
            pltpu.async_copy(tables[t].at[ixb[j].at[row]], acc[j], sG[j],
                             add=True)

        jp = (j - 1) % RING

        @pl.when(gg >= 1)
        def _():
            for _ in range(3):
                pltpu.make_async_copy(out_hbm.at[pl.ds(0, C)], acc[jp],
                                      sG[jp]).wait()
            pltpu.async_copy(acc[jp],
                             out_hbm.at[pl.ds(rbase + (gg - 1) * C, C)],
                             sO[jp])

        @pl.when(gg + RING - 1 < NCHUNK)
        def _():
            fetch_idx(gg + RING - 1, (j + RING - 1) % RING)

    def outer(g4, carry):
        for j in range(RING):
            visit(g4 * RING + j, j)
        return carry

    lax.fori_loop(0, NCHUNK // RING, outer, 0)

    jl = (NCHUNK - 1) % RING
    for _ in range(3):
        pltpu.make_async_copy(out_hbm.at[pl.ds(0, C)], acc[jl], sG[jl]).wait()
    pltpu.async_copy(acc[jl],
                     out_hbm.at[pl.ds(rbase + (NCHUNK - 1) * C, C)], sO[jl])
    for j in range(RING):
        pltpu.make_async_copy(acc[j], out_hbm.at[pl.ds(0, C)], sO[j]).wait()


def kernel(input_ids, counts, values, io_flags, positions, gas_fee,
           token_table, count_table, value_table, position_table,
           io_table, gas_table):
    t1, t2 = _build_tables(count_table, value_table, gas_table,
                           io_table, position_table)
    flat = lambda a: a.reshape(N).astype(jnp.int32)
    out = _embed_sum(token_table, t1, t2,
                     flat(input_ids), flat(counts), flat(values),
                     flat(io_flags), flat(positions), flat(gas_fee))
    return out.reshape(B, L, D)


# speedup vs baseline: 16.7848x; 1.1571x over previous
import functools

import jax
import jax.numpy as jnp
from jax import lax
from jax.experimental import pallas as pl
from jax.experimental.pallas import tpu as pltpu
from jax.experimental.pallas import tpu_sc as plsc

B, L, D = 1024, 200, 128
N = B * L
NC, NS = 2, 16
NW = NC * NS
RPW = N // NW
C = 64
NCHUNK = RPW // C
SEG = D // 16
RING = 4
CSL = C // 16

_mesh = plsc.VectorSubcoreMesh(core_axis_name="c", subcore_axis_name="s")


def _build_tables_body(cnt_ref, val_ref, gas_ref, io_ref, pos_ref,
                       t1_ref, t2_ref):
    cnt = cnt_ref[...]
    val = val_ref[...]
    gas = gas_ref[...]
    t1 = cnt[:, None, None, :] + val[None, :, None, :] + gas[None, None, :, :]
    t1 = t1.reshape(15 * 15 * 15, D)
    t1_ref[...] = jnp.broadcast_to(t1[None], (2, 3375, D)).reshape(6750, D)
    t2 = io_ref[...][:, None, :] + pos_ref[...][None, :, :]
    t2 = t2.reshape(3 * 200, D)
    t2_ref[...] = jnp.broadcast_to(t2[None], (4, 600, D)).reshape(2400, D)


def _build_tables(cnt, val, gas, io, pos):
    return pl.pallas_call(
        _build_tables_body,
        out_shape=(jax.ShapeDtypeStruct((6750, D), jnp.float32),
                   jax.ShapeDtypeStruct((2400, D), jnp.float32)),
    )(cnt, val, gas, io, pos)


@functools.partial(
    pl.kernel,
    mesh=_mesh,
    out_type=jax.ShapeDtypeStruct((N, D), jnp.float32),
    scratch_types=(
        [pltpu.VMEM((6, C), jnp.int32) for _ in range(RING)]
        + [pltpu.VMEM((C, D), jnp.float32) for _ in range(RING)]
        + [pltpu.SemaphoreType.DMA for _ in range(3 * RING)]
    ),
)
def _embed_sum(tok_t, t1_t, t2_t, itok, icnt, ival, iio, ipos, igas, out_hbm,
               ix0, ix1, ix2, ix3, ac0, ac1, ac2, ac3,
               sg0, sg1, sg2, sg3, so0, so1, so2, so3, si0, si1, si2, si3):
    idxs = [itok, icnt, ival, iio, ipos, igas]
    ixb = [ix0, ix1, ix2, ix3]
    acc = [ac0, ac1, ac2, ac3]
    sG = [sg0, sg1, sg2, sg3]
    sO = [so0, so1, so2, so3]
    sI = [si0, si1, si2, si3]
    tables = [tok_t, t1_t, t2_t]

    wid = lax.axis_index("s") * NC + lax.axis_index("c")
    rbase = wid * RPW
    zv = jnp.zeros((16,), jnp.float32)
    lane = lax.iota(jnp.int32, 16)
    r1off = (lane & 1) * 3375
    r2off = (lane & 3) * 600

    def fetch_idx(gg, j):
        for t in range(6):
            pltpu.async_copy(idxs[t].at[pl.ds(rbase + gg * C, C)],
                             ixb[j].at[t], sI[j])

    for j in range(RING - 1):
        fetch_idx(j, j)

    def visit(gg, j):
        @pl.when(gg >= RING)
        def _():
            pltpu.make_async_copy(acc[j], out_hbm.at[pl.ds(0, C)], sO[j]).wait()

        def zrow(r, z):
            for s in range(SEG):
                acc[j][r, pl.ds(s * 16, 16)] = zv
            return z
        lax.fori_loop(0, C, zrow, 0)

        for t in range(6):
            pltpu.make_async_copy(idxs[0].at[pl.ds(0, C)], ixb[j].at[t],
                                  sI[j]).wait()

        for s in range(CSL):
            cs = pl.ds(s * 16, 16)
            cv = ixb[j][1, cs] * 15 + ixb[j][2, cs]
            ixb[j][1, cs] = cv * 15 + ixb[j][5, cs] + r1off
            ixb[j][3, cs] = ixb[j][3, cs] * 200 + ixb[j][4, cs] + r2off

        for t, row in ((0, 0), (1, 1), (2, 3)):
            pltpu.async_copy(tables[t].at[ixb[j].at[row]], acc[j], sG[j],
                             add=True)

        jp = (j - 1) % RING

        @pl.when(gg >= 1)
        def _():
            for _ in range(3):
                pltpu.make_async_copy(out_hbm.at[pl.ds(0, C)], acc[jp],
                                      sG[jp]).wait()
            pltpu.async_copy(acc[jp],
                             out_hbm.at[pl.ds(rbase + (gg - 1) * C, C)],
                             sO[jp])

        @pl.when(gg + RING - 1 < NCHUNK)
        def _():
            fetch_idx(gg + RING - 1, (j + RING - 1) % RING)

    def outer(g4, carry):
        for j in range(RING):
            visit(g4 * RING + j, j)
        return carry

    lax.fori_loop(0, NCHUNK // RING, outer, 0)

    jl = (NCHUNK - 1) % RING
    for _ in range(3):
        pltpu.make_async_copy(out_hbm.at[pl.ds(0, C)], acc[jl], sG[jl]).wait()
    pltpu.async_copy(acc[jl],
                     out_hbm.at[pl.ds(rbase + (NCHUNK - 1) * C, C)], sO[jl])
    for j in range(RING):
        pltpu.make_async_copy(acc[j], out_hbm.at[pl.ds(0, C)], sO[j]).wait()


def kernel(input_ids, counts, values, io_flags, positions, gas_fee,
           token_table, count_table, value_table, position_table,
           io_table, gas_table):
    t1, t2 = _build_tables(count_table, value_table, gas_table,
                           io_table, position_table)
    flat = lambda a: a.reshape(N).astype(jnp.int32)
    out = _embed_sum(token_table, t1, t2,
                     flat(input_ids), flat(counts), flat(values),
                     flat(io_flags), flat(positions), flat(gas_fee))
    return out.reshape(B, L, D)


# speedup vs baseline: 17.3857x; 1.0358x over previous
import functools

import jax
import jax.numpy as jnp
from jax import lax
from jax.experimental import pallas as pl
from jax.experimental.pallas import tpu as pltpu
from jax.experimental.pallas import tpu_sc as plsc

B, L, D = 1024, 200, 128
N = B * L
NC, NS = 2, 16
NW = NC * NS
RPW = N // NW
C = 64
NCHUNK = RPW // C
SEG = D // 16
RING = 4
CSL = C // 16

_mesh = plsc.VectorSubcoreMesh(core_axis_name="c", subcore_axis_name="s")


def _build_tables_body(cnt_ref, val_ref, gas_ref, io_ref, pos_ref,
                       t1_ref, t2_ref):
    cnt = cnt_ref[...]
    val = val_ref[...]
    gas = gas_ref[...]
    t1 = cnt[:, None, None, :] + val[None, :, None, :] + gas[None, None, :, :]
    t1 = t1.reshape(15 * 15 * 15, D)
    t1_ref[...] = jnp.broadcast_to(t1[None], (4, 3375, D)).reshape(13500, D)
    t2 = io_ref[...][:, None, :] + pos_ref[...][None, :, :]
    t2 = t2.reshape(3 * 200, D)
    t2_ref[...] = jnp.broadcast_to(t2[None], (8, 600, D)).reshape(4800, D)


def _build_tables(cnt, val, gas, io, pos):
    return pl.pallas_call(
        _build_tables_body,
        out_shape=(jax.ShapeDtypeStruct((13500, D), jnp.float32),
                   jax.ShapeDtypeStruct((4800, D), jnp.float32)),
    )(cnt, val, gas, io, pos)


@functools.partial(
    pl.kernel,
    mesh=_mesh,
    out_type=jax.ShapeDtypeStruct((N, D), jnp.float32),
    scratch_types=(
        [pltpu.VMEM((6, C), jnp.int32) for _ in range(RING)]
        + [pltpu.VMEM((C, D), jnp.float32) for _ in range(RING)]
        + [pltpu.SemaphoreType.DMA for _ in range(3 * RING)]
    ),
)
def _embed_sum(tok_t, t1_t, t2_t, itok, icnt, ival, iio, ipos, igas, out_hbm,
               ix0, ix1, ix2, ix3, ac0, ac1, ac2, ac3,
               sg0, sg1, sg2, sg3, so0, so1, so2, so3, si0, si1, si2, si3):
    idxs = [itok, icnt, ival, iio, ipos, igas]
    ixb = [ix0, ix1, ix2, ix3]
    acc = [ac0, ac1, ac2, ac3]
    sG = [sg0, sg1, sg2, sg3]
    sO = [so0, so1, so2, so3]
    sI = [si0, si1, si2, si3]
    tables = [tok_t, t1_t, t2_t]

    wid = lax.axis_index("s") * NC + lax.axis_index("c")
    rbase = wid * RPW
    zv = jnp.zeros((16,), jnp.float32)
    lane = lax.iota(jnp.int32, 16)
    r1off = (lane & 3) * 3375
    r2off = (lane & 7) * 600

    def fetch_idx(gg, j):
        for t in range(6):
            pltpu.async_copy(idxs[t].at[pl.ds(rbase + gg * C, C)],
                             ixb[j].at[t], sI[j])

    for j in range(RING - 1):
        fetch_idx(j, j)

    def visit(gg, j):
        @pl.when(gg >= RING)
        def _():
            pltpu.make_async_copy(acc[j], out_hbm.at[pl.ds(0, C)], sO[j]).wait()

        def zrow(r, z):
            for s in range(SEG):
                acc[j][r, pl.ds(s * 16, 16)] = zv
            return z
        lax.fori_loop(0, C, zrow, 0)

        for t in range(6):
            pltpu.make_async_copy(idxs[0].at[pl.ds(0, C)], ixb[j].at[t],
                                  sI[j]).wait()

        for s in range(CSL):
            cs = pl.ds(s * 16, 16)
            cv = ixb[j][1, cs] * 15 + ixb[j][2, cs]
            ixb[j][1, cs] = cv * 15 + ixb[j][5, cs] + r1off
            ixb[j][3, cs] = ixb[j][3, cs] * 200 + ixb[j][4, cs] + r2off

        for t, row in ((0, 0), (1, 1), (2, 3)):
            pltpu.async_copy(tables[t].at[ixb[j].at[row]], acc[j], sG[j],
                             add=True)

        jp = (j - 1) % RING

        @pl.when(gg >= 1)
        def _():
            for _ in range(3):
                pltpu.make_async_copy(out_hbm.at[pl.ds(0, C)], acc[jp],
                                      sG[jp]).wait()
            pltpu.async_copy(acc[jp],
                             out_hbm.at[pl.ds(rbase + (gg - 1) * C, C)],
                             sO[jp])

        @pl.when(gg + RING - 1 < NCHUNK)
        def _():
            fetch_idx(gg + RING - 1, (j + RING - 1) % RING)

    def outer(g4, carry):
        for j in range(RING):
            visit(g4 * RING + j, j)
        return carry

    lax.fori_loop(0, NCHUNK // RING, outer, 0)

    jl = (NCHUNK - 1) % RING
    for _ in range(3):
        pltpu.make_async_copy(out_hbm.at[pl.ds(0, C)], acc[jl], sG[jl]).wait()
    pltpu.async_copy(acc[jl],
                     out_hbm.at[pl.ds(rbase + (NCHUNK - 1) * C, C)], sO[jl])
    for j in range(RING):
        pltpu.make_async_copy(acc[j], out_hbm.at[pl.ds(0, C)], sO[j]).wait()


def kernel(input_ids, counts, values, io_flags, positions, gas_fee,
           token_table, count_table, value_table, position_table,
           io_table, gas_table):
    t1, t2 = _build_tables(count_table, value_table, gas_table,
                           io_table, position_table)
    flat = lambda a: a.reshape(N).astype(jnp.int32)
    out = _embed_sum(token_table, t1, t2,
                     flat(input_ids), flat(counts), flat(values),
                     flat(io_flags), flat(positions), flat(gas_fee))
    return out.reshape(B, L, D)
